# baseline (speedup 1.0000x reference)
import jax
import jax.numpy as jnp
from jax import lax
from jax.experimental import pallas as pl
from jax.experimental.pallas import tpu as pltpu
from jax.experimental.pallas import tpu_sc as plsc

NC, NS = 2, 16
NW = NC * NS
BATCH, SEQ, D = 4096, 26, 64
BBLK = BATCH // NW

_mesh = plsc.VectorSubcoreMesh(
    core_axis_name="c", subcore_axis_name="s", num_cores=NC, num_subcores=NS
)


def _gather_body(ids_hbm, table_hbm, out_hbm, idx_v, rows_v, gsem0, gsem1):
    wid = lax.axis_index("s") * NC + lax.axis_index("c")
    b0 = wid * BBLK
    pltpu.sync_copy(ids_hbm.at[:, pl.ds(b0, BBLK)], idx_v)

    def fire(s, slot, sem):
        pltpu.async_copy(table_hbm.at[idx_v.at[s]], rows_v.at[slot], sem)

    def drain_store(s, slot, sem):
        pltpu.make_async_copy(
            table_hbm.at[idx_v.at[s]], rows_v.at[slot], sem
        ).wait()
        pltpu.sync_copy(
            rows_v.at[slot], out_hbm.at[pl.ds(b0, BBLK), s, pl.ds(0, D)]
        )

    fire(0, 0, gsem0)

    def body(h, carry):
        s0 = 2 * h
        fire(s0 + 1, 1, gsem1)
        drain_store(s0, 0, gsem0)

        @pl.when(h + 1 < SEQ // 2)
        def _():
            fire(s0 + 2, 0, gsem0)

        drain_store(s0 + 1, 1, gsem1)
        return carry

    lax.fori_loop(0, SEQ // 2, body, 0)


_gather = pl.kernel(
    _gather_body,
    out_type=jax.ShapeDtypeStruct((BATCH, 32, 128), jnp.float32),
    mesh=_mesh,
    scratch_types=[
        pltpu.VMEM((SEQ, BBLK), jnp.int32),
        pltpu.VMEM((2, BBLK, D), jnp.float32),
        pltpu.SemaphoreType.DMA,
        pltpu.SemaphoreType.DMA,
    ],
    compiler_params=pltpu.CompilerParams(use_tc_tiling_on_sc=False),
)


@jax.jit
def kernel(input_ids, table):
    ids_t = input_ids.astype(jnp.int32).T
    out = _gather(ids_t, table)
    return out[:, :SEQ, :D]


# speedup vs baseline: 1.0028x; 1.0028x over previous
import jax
import jax.numpy as jnp
from jax import lax
from jax.experimental import pallas as pl
from jax.experimental.pallas import tpu as pltpu
from jax.experimental.pallas import tpu_sc as plsc

NC, NS = 2, 16
NW = NC * NS
BATCH, SEQ, D = 4096, 26, 64
BBLK = BATCH // NW

_mesh = plsc.VectorSubcoreMesh(
    core_axis_name="c", subcore_axis_name="s", num_cores=NC, num_subcores=NS
)


def _gather_body(ids_hbm, table_hbm, out_hbm, idx_v, rows_v, gsem0, gsem1):
    wid = lax.axis_index("s") * NC + lax.axis_index("c")
    b0 = wid * BBLK
    pltpu.sync_copy(ids_hbm.at[:, pl.ds(b0, BBLK)], idx_v)

    def fire(s, slot, sem):
        pltpu.async_copy(table_hbm.at[idx_v.at[s]], rows_v.at[slot], sem)

    def drain_store(s, slot, sem):
        pltpu.make_async_copy(
            table_hbm.at[idx_v.at[s]], rows_v.at[slot], sem
        ).wait()
        pltpu.sync_copy(
            rows_v.at[slot], out_hbm.at[pl.ds(b0, BBLK), s, pl.ds(0, D)]
        )

    fire(0, 0, gsem0)

    def body(h, carry):
        s0 = 2 * h
        fire(s0 + 1, 1, gsem1)
        drain_store(s0, 0, gsem0)

        @pl.when(h + 1 < SEQ // 2)
        def _():
            fire(s0 + 2, 0, gsem0)

        drain_store(s0 + 1, 1, gsem1)
        return carry

    lax.fori_loop(0, SEQ // 2, body, 0)


_gather = pl.kernel(
    _gather_body,
    out_type=jax.ShapeDtypeStruct((BATCH, 32, 128), jnp.float32),
    mesh=_mesh,
    scratch_types=[
        pltpu.VMEM((SEQ, BBLK), jnp.int32),
        pltpu.VMEM((2, BBLK, D), jnp.float32),
        pltpu.SemaphoreType.DMA,
        pltpu.SemaphoreType.DMA,
    ],
    compiler_params=pltpu.CompilerParams(
        use_tc_tiling_on_sc=False,
        disable_bounds_checks=True,
        disable_semaphore_checks=True,
    ),
)


@jax.jit
def kernel(input_ids, table):
    ids_t = input_ids.astype(jnp.int32).T
    out = _gather(ids_t, table)
    return out[:, :SEQ, :D]
